# baseline (speedup 1.0000x reference)
import jax
import jax.numpy as jnp
from jax import lax
from jax.experimental import pallas as pl
from jax.experimental.pallas import tpu as pltpu
from jax.experimental.pallas import tpu_sc as plsc

_B, _IN, _O1, _O2, _F = 1024, 1024, 2048, 2048, 32
_NC, _NS, _L = 2, 16, 16
_NW = _NC * _NS
_R = _O1 // _NW
_CH = 16


def _densify_body(idx1_hbm, w1_hbm, idx2_hbm, w2_hbm, d1_hbm, d2_hbm,
                  idx1_v, w1_v, idx2_v, w2_v, acc1, acc2, sem_a, sem_b):
    wid = lax.axis_index("s") * _NC + lax.axis_index("c")
    lane = lax.iota(jnp.int32, _L)
    zero = jnp.zeros((_L,), jnp.float32)
    base = wid * _R

    pltpu.sync_copy(idx1_hbm.at[pl.ds(base * _F, _R * _F)], idx1_v)
    pltpu.sync_copy(w1_hbm.at[pl.ds(base * _F, _R * _F)], w1_v)
    pltpu.sync_copy(idx2_hbm.at[pl.ds(base * _F, _R * _F)], idx2_v)
    pltpu.sync_copy(w2_hbm.at[pl.ds(base * _F, _R * _F)], w2_v)

    sems = (sem_a, sem_b)

    def layer(idx_v, w_v, acc, d_hbm, ncols, offset):
        pend = [None, None]
        for c in range(_R // _CH):
            buf = c % 2
            if pend[buf] is not None:
                pend[buf].wait()

            def zbody(j, carry):
                for r in range(_CH):
                    for k in range(2):
                        acc[buf, r, pl.ds((j * 2 + k) * _L, _L)] = zero
                return carry

            lax.fori_loop(0, ncols // (2 * _L), zbody, 0)
            for f in range(_F):
                src = (lane + c * _CH) * _F + f
                col = plsc.load_gather(idx_v, [src]) - offset
                wv = plsc.load_gather(w_v, [src])
                plsc.addupdate_scatter(acc.at[buf], [lane, col], wv)
            pend[buf] = pltpu.async_copy(
                acc.at[buf], d_hbm.at[pl.ds(base + c * _CH, _CH)], sems[buf])
        for p in pend:
            if p is not None:
                p.wait()

    layer(idx1_v, w1_v, acc1, d1_hbm, _IN, 1)
    layer(idx2_v, w2_v, acc2, d2_hbm, _O1, 1 + _IN)


_densify = pl.kernel(
    _densify_body,
    out_type=[
        jax.ShapeDtypeStruct((_O1, _IN), jnp.float32),
        jax.ShapeDtypeStruct((_O2, _O1), jnp.float32),
    ],
    mesh=plsc.VectorSubcoreMesh(core_axis_name="c", subcore_axis_name="s"),
    compiler_params=pltpu.CompilerParams(needs_layout_passes=False),
    scratch_types=[
        pltpu.VMEM((_R * _F,), jnp.int32),
        pltpu.VMEM((_R * _F,), jnp.float32),
        pltpu.VMEM((_R * _F,), jnp.int32),
        pltpu.VMEM((_R * _F,), jnp.float32),
        pltpu.VMEM((2, _CH, _IN), jnp.float32),
        pltpu.VMEM((2, _CH, _O1), jnp.float32),
        pltpu.SemaphoreType.DMA,
        pltpu.SemaphoreType.DMA,
    ],
)


def _mm_body(x_ref, d1_hbm, d2_hbm, b1_ref, b2_ref, o_ref,
             d1_v, d2_v, sem1, sem2):
    i = pl.program_id(0)
    c1 = pltpu.make_async_copy(d1_hbm, d1_v, sem1)
    c2 = pltpu.make_async_copy(d2_hbm, d2_v, sem2)

    @pl.when(i == 0)
    def _():
        c1.start()
        c2.start()

    @pl.when(i == 0)
    def _():
        c1.wait()

    h = lax.dot_general(x_ref[...], d1_v[...], (((1,), (1,)), ((), ())),
                        preferred_element_type=jnp.float32)
    h = jnp.maximum(h + b1_ref[...], 0.0)

    @pl.when(i == 0)
    def _():
        c2.wait()

    y = lax.dot_general(h, d2_v[...], (((1,), (1,)), ((), ())),
                        preferred_element_type=jnp.float32)
    o_ref[...] = y + b2_ref[...]


_BM = 256

_mm = pl.pallas_call(
    _mm_body,
    grid=(_B // _BM,),
    in_specs=[
        pl.BlockSpec((_BM, _IN), lambda i: (i, 0)),
        pl.BlockSpec(memory_space=pl.ANY),
        pl.BlockSpec(memory_space=pl.ANY),
        pl.BlockSpec((1, _O1), lambda i: (0, 0)),
        pl.BlockSpec((1, _O2), lambda i: (0, 0)),
    ],
    out_specs=pl.BlockSpec((_BM, _O2), lambda i: (i, 0)),
    out_shape=jax.ShapeDtypeStruct((_B, _O2), jnp.float32),
    scratch_shapes=[
        pltpu.VMEM((_O1, _IN), jnp.float32),
        pltpu.VMEM((_O2, _O1), jnp.float32),
        pltpu.SemaphoreType.DMA,
        pltpu.SemaphoreType.DMA,
    ],
)


def kernel(x, W1, b1, W2, b2, idx1, idx2, out_idx1, out_idx2):
    d1, d2 = _densify(idx1.reshape(-1), W1.reshape(-1),
                      idx2.reshape(-1), W2.reshape(-1))
    return _mm(x, d1, d2, b1.reshape(1, _O1), b2.reshape(1, _O2))


# speedup vs baseline: 1.0432x; 1.0432x over previous
import jax
import jax.numpy as jnp
from jax import lax
from jax.experimental import pallas as pl
from jax.experimental.pallas import tpu as pltpu
from jax.experimental.pallas import tpu_sc as plsc

_B, _IN, _O1, _O2, _F = 1024, 1024, 2048, 2048, 32
_NC, _NS, _L = 2, 16, 16
_NW = _NC * _NS
_R = _O1 // _NW
_CH = 16


def _make_densify(nrows, ncols, offset):
    def body(idx_hbm, w_hbm, d_hbm, idx_v, w_v, acc, sem_a, sem_b):
        wid = lax.axis_index("s") * _NC + lax.axis_index("c")
        lane = lax.iota(jnp.int32, _L)
        zero = jnp.zeros((_L,), jnp.float32)
        base = wid * _R

        pltpu.sync_copy(idx_hbm.at[pl.ds(base * _F, _R * _F)], idx_v)
        pltpu.sync_copy(w_hbm.at[pl.ds(base * _F, _R * _F)], w_v)

        sems = (sem_a, sem_b)
        pend = [None, None]
        for c in range(_R // _CH):
            buf = c % 2
            if pend[buf] is not None:
                pend[buf].wait()

            def zbody(j, carry):
                for r in range(_CH):
                    for k in range(2):
                        acc[buf, r, pl.ds((j * 2 + k) * _L, _L)] = zero
                return carry

            lax.fori_loop(0, ncols // (2 * _L), zbody, 0)
            for f in range(_F):
                src = (lane + c * _CH) * _F + f
                col = plsc.load_gather(idx_v, [src]) - offset
                wv = plsc.load_gather(w_v, [src])
                plsc.addupdate_scatter(acc.at[buf], [lane, col], wv)
            pend[buf] = pltpu.async_copy(
                acc.at[buf], d_hbm.at[pl.ds(base + c * _CH, _CH)], sems[buf])
        for p in pend:
            if p is not None:
                p.wait()

    return pl.kernel(
        body,
        out_type=jax.ShapeDtypeStruct((nrows, ncols), jnp.float32),
        mesh=plsc.VectorSubcoreMesh(core_axis_name="c", subcore_axis_name="s"),
        compiler_params=pltpu.CompilerParams(needs_layout_passes=False),
        scratch_types=[
            pltpu.VMEM((_R * _F,), jnp.int32),
            pltpu.VMEM((_R * _F,), jnp.float32),
            pltpu.VMEM((2, _CH, ncols), jnp.float32),
            pltpu.SemaphoreType.DMA,
            pltpu.SemaphoreType.DMA,
        ],
    )


_densify1 = _make_densify(_O1, _IN, 1)
_densify2 = _make_densify(_O2, _O1, 1 + _IN)


def _make_mm(bm, n, k, relu):
    def body(x_ref, d_hbm, b_ref, o_ref, d_v, sem):
        i = pl.program_id(0)
        cp = pltpu.make_async_copy(d_hbm, d_v, sem)

        @pl.when(i == 0)
        def _():
            cp.start()
            cp.wait()

        y = lax.dot_general(x_ref[...], d_v[...], (((1,), (1,)), ((), ())),
                            preferred_element_type=jnp.float32)
        y = y + b_ref[...]
        if relu:
            y = jnp.maximum(y, 0.0)
        o_ref[...] = y

    return pl.pallas_call(
        body,
        grid=(_B // bm,),
        in_specs=[
            pl.BlockSpec((bm, k), lambda i: (i, 0)),
            pl.BlockSpec(memory_space=pl.ANY),
            pl.BlockSpec((1, n), lambda i: (0, 0)),
        ],
        out_specs=pl.BlockSpec((bm, n), lambda i: (i, 0)),
        out_shape=jax.ShapeDtypeStruct((_B, n), jnp.float32),
        scratch_shapes=[
            pltpu.VMEM((n, k), jnp.float32),
            pltpu.SemaphoreType.DMA,
        ],
    )


_mm1 = _make_mm(256, _O1, _IN, True)
_mm2 = _make_mm(256, _O2, _O1, False)


def kernel(x, W1, b1, W2, b2, idx1, idx2, out_idx1, out_idx2):
    d1 = _densify1(idx1.reshape(-1), W1.reshape(-1))
    d2 = _densify2(idx2.reshape(-1), W2.reshape(-1))
    h = _mm1(x, d1, b1.reshape(1, _O1))
    return _mm2(h, d2, b2.reshape(1, _O2))
